# Initial kernel scaffold; baseline (speedup 1.0000x reference)
import jax
import jax.numpy as jnp
from jax.experimental import pallas as pl


def kernel(inputs, embedding):
    raise NotImplementedError("write your pallas kernel here")


# speedup vs baseline: 2.6260x; 2.6260x over previous
import jax
import jax.numpy as jnp
from jax import lax
from jax.experimental import pallas as pl
from jax.experimental.pallas import tpu as pltpu
from jax.experimental.pallas import tpu_sc as plsc

B = 32
HW = 56 * 56
NCH = 3
VOC = 1024
D = 128
DSEG = 32
DBLK = 8


def _tr_body(emb_ref, out_ref):
    out_ref[...] = jnp.transpose(emb_ref[...], (0, 2, 1))


@jax.jit
def _transpose_table(embedding):
    emb3 = embedding.reshape(NCH, VOC, D)
    out = pl.pallas_call(
        _tr_body,
        out_shape=jax.ShapeDtypeStruct((NCH, D, VOC), jnp.float32),
        grid=(NCH,),
        in_specs=[pl.BlockSpec((1, VOC, D), lambda i: (i, 0, 0))],
        out_specs=pl.BlockSpec((1, D, VOC), lambda i: (i, 0, 0)),
    )(emb3)
    return out.reshape(NCH * D * VOC)


def _sc_body(in_hbm, tab_hbm, out_hbm, idxb, idxc, seg, st0, st1, sm0, sm1):
    cid = lax.axis_index("c")
    sid = lax.axis_index("s")
    b = sid * 2 + cid

    pltpu.sync_copy(in_hbm.at[pl.ds(b * (HW * NCH), HW * NCH)], idxb)

    for c in range(NCH):
        def build(p16, carry):
            base = p16 * 16
            iidx = (lax.iota(jnp.int32, 16) + base) * NCH + c
            idxc[pl.ds(base, 16)] = plsc.load_gather(idxb, [iidx])
            return carry

        lax.fori_loop(0, HW // 16, build, 0)

        for dseg in range(D // DSEG):
            pltpu.sync_copy(
                tab_hbm.at[pl.ds((c * D + dseg * DSEG) * VOC, DSEG * VOC)],
                seg)

            for db in range(DSEG // DBLK):
                blk = dseg * (DSEG // DBLK) + db
                stage = st0 if blk % 2 == 0 else st1
                sem = sm0 if blk % 2 == 0 else sm1
                row0 = c * D + blk * DBLK

                def out_dst(r0):
                    return out_hbm.at[b, pl.ds(r0, DBLK), :]

                if c * (D // DBLK) + blk >= 2:
                    pltpu.make_async_copy(stage, out_dst(row0 - 2 * DBLK),
                                          sem).wait()

                def fill(p16, carry):
                    iv = idxc[pl.ds(p16 * 16, 16)]
                    for dloc in range(DBLK):
                        g = plsc.load_gather(
                            seg, [iv + (db * DBLK + dloc) * VOC])
                        stage[dloc, pl.ds(p16 * 16, 16)] = g
                    return carry

                lax.fori_loop(0, HW // 16, fill, 0)

                pltpu.async_copy(stage, out_dst(row0), sem)

    for blk in (D // DBLK - 2, D // DBLK - 1):
        stage = st0 if blk % 2 == 0 else st1
        sem = sm0 if blk % 2 == 0 else sm1
        row0 = (NCH - 1) * D + blk * DBLK
        pltpu.make_async_copy(stage, out_hbm.at[b, pl.ds(row0, DBLK), :],
                              sem).wait()


@jax.jit
def _sc_call(flat_in, tab_flat):
    mesh = plsc.VectorSubcoreMesh(core_axis_name="c", subcore_axis_name="s")
    f = pl.kernel(
        _sc_body,
        out_type=jax.ShapeDtypeStruct((B, NCH * D, HW), jnp.float32),
        mesh=mesh,
        scratch_types=[
            pltpu.VMEM((HW * NCH,), jnp.int32),
            pltpu.VMEM((HW,), jnp.int32),
            pltpu.VMEM((DSEG * VOC,), jnp.float32),
            pltpu.VMEM((DBLK, HW), jnp.float32),
            pltpu.VMEM((DBLK, HW), jnp.float32),
            pltpu.SemaphoreType.DMA,
            pltpu.SemaphoreType.DMA,
        ],
        compiler_params=pltpu.CompilerParams(needs_layout_passes=False),
    )
    return f(flat_in, tab_flat)


def kernel(inputs, embedding):
    tab_flat = _transpose_table(embedding)
    flat_in = inputs.reshape(B * HW * NCH)
    out3 = _sc_call(flat_in, tab_flat)
    return out3.reshape(B, NCH * D, 56, 56)


# speedup vs baseline: 4.7391x; 1.8047x over previous
import jax
import jax.numpy as jnp
from jax import lax
from jax.experimental import pallas as pl
from jax.experimental.pallas import tpu as pltpu
from jax.experimental.pallas import tpu_sc as plsc

B = 32
HW = 56 * 56
NCH = 3
VOC = 1024
D = 128
DSEG = 32
DBLK = 8


def _tr_body(emb_ref, out_ref):
    out_ref[...] = jnp.transpose(emb_ref[...], (0, 2, 1))


@jax.jit
def _transpose_table(embedding):
    emb3 = embedding.reshape(NCH, VOC, D)
    out = pl.pallas_call(
        _tr_body,
        out_shape=jax.ShapeDtypeStruct((NCH, D, VOC), jnp.float32),
        grid=(NCH,),
        in_specs=[pl.BlockSpec((1, VOC, D), lambda i: (i, 0, 0))],
        out_specs=pl.BlockSpec((1, D, VOC), lambda i: (i, 0, 0)),
    )(emb3)
    return out.reshape(NCH * D * VOC)


def _sc_body(in_hbm, tab_hbm, out_hbm, idxb, idxc, seg, st0, st1, sm0, sm1):
    cid = lax.axis_index("c")
    sid = lax.axis_index("s")
    b = sid * 2 + cid

    pltpu.sync_copy(in_hbm.at[pl.ds(b * (HW * NCH), HW * NCH)], idxb)

    for c in range(NCH):
        @plsc.parallel_loop(0, HW // 16, 1, unroll=2)
        def build(p16):
            base = p16 * 16
            iidx = (lax.iota(jnp.int32, 16) + base) * NCH + c
            idxc[pl.ds(base, 16)] = plsc.load_gather(idxb, [iidx])

        for dseg in range(D // DSEG):
            pltpu.sync_copy(
                tab_hbm.at[pl.ds((c * D + dseg * DSEG) * VOC, DSEG * VOC)],
                seg)

            for db in range(DSEG // DBLK):
                blk = dseg * (DSEG // DBLK) + db
                stage = st0 if blk % 2 == 0 else st1
                sem = sm0 if blk % 2 == 0 else sm1
                row0 = c * D + blk * DBLK

                def out_dst(r0):
                    return out_hbm.at[b, pl.ds(r0, DBLK), :]

                if c * (D // DBLK) + blk >= 2:
                    pltpu.make_async_copy(stage, out_dst(row0 - 2 * DBLK),
                                          sem).wait()

                @plsc.parallel_loop(0, HW // 16, 1, unroll=2)
                def fill(p16):
                    iv = idxc[pl.ds(p16 * 16, 16)]
                    for dloc in range(DBLK):
                        g = plsc.load_gather(
                            seg, [iv + (db * DBLK + dloc) * VOC])
                        stage[dloc, pl.ds(p16 * 16, 16)] = g

                pltpu.async_copy(stage, out_dst(row0), sem)

    for blk in (D // DBLK - 2, D // DBLK - 1):
        stage = st0 if blk % 2 == 0 else st1
        sem = sm0 if blk % 2 == 0 else sm1
        row0 = (NCH - 1) * D + blk * DBLK
        pltpu.make_async_copy(stage, out_hbm.at[b, pl.ds(row0, DBLK), :],
                              sem).wait()


@jax.jit
def _sc_call(flat_in, tab_flat):
    mesh = plsc.VectorSubcoreMesh(core_axis_name="c", subcore_axis_name="s")
    f = pl.kernel(
        _sc_body,
        out_type=jax.ShapeDtypeStruct((B, NCH * D, HW), jnp.float32),
        mesh=mesh,
        scratch_types=[
            pltpu.VMEM((HW * NCH,), jnp.int32),
            pltpu.VMEM((HW,), jnp.int32),
            pltpu.VMEM((DSEG * VOC,), jnp.float32),
            pltpu.VMEM((DBLK, HW), jnp.float32),
            pltpu.VMEM((DBLK, HW), jnp.float32),
            pltpu.SemaphoreType.DMA,
            pltpu.SemaphoreType.DMA,
        ],
        compiler_params=pltpu.CompilerParams(needs_layout_passes=False),
    )
    return f(flat_in, tab_flat)


def kernel(inputs, embedding):
    tab_flat = _transpose_table(embedding)
    flat_in = inputs.reshape(B * HW * NCH)
    out3 = _sc_call(flat_in, tab_flat)
    return out3.reshape(B, NCH * D, 56, 56)
